# Initial kernel scaffold; baseline (speedup 1.0000x reference)
import jax
import jax.numpy as jnp
from jax.experimental import pallas as pl


def kernel(loc_pred, conf_pred, anchors, targets):
    raise NotImplementedError("write your pallas kernel here")


# speedup vs baseline: 29.1489x; 29.1489x over previous
import functools

import jax
import jax.numpy as jnp
from jax.experimental import pallas as pl
from jax.experimental.pallas import tpu as pltpu

NEG_POS_RATIO = 3
OVERLAP_THRESH = 0.5
V0, V1 = 0.1, 0.2
AR, AC = 512, 128
NOBJ = 16


def _loss_kernel(tgt_ref, anch_ref, lp_ref, cp_ref, out_ref):
    a_cx = anch_ref[0]
    a_w = anch_ref[1]
    a_lo = a_cx - a_w / 2.0
    a_hi = a_cx + a_w / 2.0
    len_a = a_hi - a_lo

    rows = jax.lax.broadcasted_iota(jnp.int32, (AR, AC), 0)
    cols = jax.lax.broadcasted_iota(jnp.int32, (AR, AC), 1)
    lin = rows * AC + cols

    t_lo = [tgt_ref[0, j, 0] for j in range(NOBJ)]
    t_hi = [tgt_ref[0, j, 1] for j in range(NOBJ)]

    bto = jnp.zeros((AR, AC), jnp.float32)
    bti = jnp.zeros((AR, AC), jnp.int32)
    bpi = []
    for j in range(NOBJ):
        inter = jnp.maximum(
            jnp.minimum(t_hi[j], a_hi) - jnp.maximum(t_lo[j], a_lo), 0.0)
        union = jnp.maximum((t_hi[j] - t_lo[j]) + len_a - inter, 1e-10)
        iou = inter / union
        mx = jnp.max(iou)
        bpi.append(jnp.min(jnp.where(iou == mx, lin, AR * AC)))
        upd = iou > bto
        bto = jnp.where(upd, iou, bto)
        bti = jnp.where(upd, j, bti)

    forced = jnp.zeros((AR, AC), jnp.bool_)
    for j in range(NOBJ):
        hit = lin == bpi[j]
        forced = forced | hit
        bti = jnp.where(hit, j, bti)
    pos = forced | (bto >= OVERLAP_THRESH)
    posf = pos.astype(jnp.float32)

    m_lo = jnp.zeros((AR, AC), jnp.float32)
    m_hi = jnp.zeros((AR, AC), jnp.float32)
    for j in range(NOBJ):
        sel = bti == j
        m_lo = jnp.where(sel, t_lo[j], m_lo)
        m_hi = jnp.where(sel, t_hi[j], m_hi)

    g_cx = ((m_lo + m_hi) / 2.0 - a_cx) / (V0 * a_w)
    g_w = jnp.log(jnp.maximum((m_hi - m_lo) / a_w, 1e-10)) / V1
    d0 = lp_ref[0, 0] - g_cx
    d1 = lp_ref[0, 1] - g_w
    ad0 = jnp.abs(d0)
    ad1 = jnp.abs(d1)
    sl1 = (jnp.where(ad0 < 1.0, 0.5 * d0 * d0, ad0 - 0.5) +
           jnp.where(ad1 < 1.0, 0.5 * d1 * d1, ad1 - 0.5))
    loss_l = jnp.sum(sl1 * posf)

    x0 = cp_ref[0, 0]
    x1 = cp_ref[0, 1]
    m01 = jnp.maximum(x0, x1)
    lse = m01 + jnp.log(jnp.exp(x0 - m01) + jnp.exp(x1 - m01))
    ce = lse - jnp.where(pos, x1, x0)
    ce_pos = jnp.sum(jnp.where(pos, ce, 0.0))

    num_pos = jnp.sum(pos.astype(jnp.int32))
    num_neg = jnp.minimum(NEG_POS_RATIO * num_pos, AR * AC - num_pos)

    proxy = jnp.where(pos, 0.0, ce)
    bits = jax.lax.bitcast_convert_type(proxy, jnp.int32)

    def bs_body(_, lo_hi):
        lo, hi = lo_hi
        mid = lo + (hi - lo) // 2
        cnt = jnp.sum((bits > mid).astype(jnp.int32))
        take = cnt >= num_neg
        return (jnp.where(take, mid, lo), jnp.where(take, hi, mid))

    lo0 = jnp.int32(0)
    hi0 = jnp.int32(2**31 - 1)
    _, kth = jax.lax.fori_loop(0, 31, bs_body, (lo0, hi0))
    kth_val = jax.lax.bitcast_convert_type(kth, jnp.float32)
    gt = bits > kth
    sum_gt = jnp.sum(jnp.where(gt, proxy, 0.0))
    cnt_gt = jnp.sum(gt.astype(jnp.int32))
    topk = sum_gt + (num_neg - cnt_gt).astype(jnp.float32) * kth_val

    lane = jax.lax.broadcasted_iota(jnp.int32, (1, 128), 1)
    row = (jnp.where(lane == 0, loss_l, 0.0) +
           jnp.where(lane == 1, ce_pos, 0.0) +
           jnp.where(lane == 2, topk, 0.0) +
           jnp.where(lane == 3, num_pos.astype(jnp.float32), 0.0))
    out_ref[...] = row.reshape(1, 1, 128)


@jax.jit
def kernel(loc_pred, conf_pred, anchors, targets):
    B, A, _ = loc_pred.shape
    lp = loc_pred.transpose(0, 2, 1).reshape(B, 2, AR, AC)
    cp = conf_pred.transpose(0, 2, 1).reshape(B, 2, AR, AC)
    anch = anchors.T.reshape(2, AR, AC)

    parts = pl.pallas_call(
        _loss_kernel,
        grid=(B,),
        in_specs=[
            pl.BlockSpec((1, NOBJ, 3), lambda b: (b, 0, 0),
                         memory_space=pltpu.SMEM),
            pl.BlockSpec((2, AR, AC), lambda b: (0, 0, 0)),
            pl.BlockSpec((1, 2, AR, AC), lambda b: (b, 0, 0, 0)),
            pl.BlockSpec((1, 2, AR, AC), lambda b: (b, 0, 0, 0)),
        ],
        out_specs=pl.BlockSpec((1, 1, 128), lambda b: (b, 0, 0)),
        out_shape=jax.ShapeDtypeStruct((B, 1, 128), jnp.float32),
    )(targets, anch, lp, cp)

    loss_l = jnp.sum(parts[:, 0, 0])
    loss_c = jnp.sum(parts[:, 0, 1] + parts[:, 0, 2])
    total = jnp.sum(parts[:, 0, 3])
    return (loss_l / total, loss_c / total)


# speedup vs baseline: 44.8369x; 1.5382x over previous
import functools

import jax
import jax.numpy as jnp
from jax.experimental import pallas as pl
from jax.experimental.pallas import tpu as pltpu

NEG_POS_RATIO = 3
OVERLAP_THRESH = 0.5
V0, V1 = 0.1, 0.2
AR, AC = 512, 128
NOBJ = 16


def _loss_kernel(tgt_ref, anch_ref, lp_ref, cp_ref, out_ref):
    a_cx = anch_ref[0]
    a_w = anch_ref[1]
    a_lo = a_cx - a_w / 2.0
    a_hi = a_cx + a_w / 2.0
    len_a = a_hi - a_lo

    rows = jax.lax.broadcasted_iota(jnp.int32, (AR, AC), 0)
    cols = jax.lax.broadcasted_iota(jnp.int32, (AR, AC), 1)
    lin = rows * AC + cols

    t_lo = [tgt_ref[0, j, 0] for j in range(NOBJ)]
    t_hi = [tgt_ref[0, j, 1] for j in range(NOBJ)]

    bto = jnp.zeros((AR, AC), jnp.float32)
    m_lo = jnp.full((AR, AC), t_lo[0], jnp.float32)
    m_hi = jnp.full((AR, AC), t_hi[0], jnp.float32)
    bpi = []
    for j in range(NOBJ):
        inter = jnp.maximum(
            jnp.minimum(t_hi[j], a_hi) - jnp.maximum(t_lo[j], a_lo), 0.0)
        union = (t_hi[j] - t_lo[j]) + len_a - inter
        iou = inter / union
        mx = jnp.max(iou)
        bpi.append(jnp.min(jnp.where(iou == mx, lin, AR * AC)))
        upd = iou > bto
        bto = jnp.where(upd, iou, bto)
        m_lo = jnp.where(upd, t_lo[j], m_lo)
        m_hi = jnp.where(upd, t_hi[j], m_hi)

    forced = jnp.zeros((AR, AC), jnp.bool_)
    for j in range(NOBJ):
        hit = lin == bpi[j]
        forced = forced | hit
        m_lo = jnp.where(hit, t_lo[j], m_lo)
        m_hi = jnp.where(hit, t_hi[j], m_hi)
    pos = forced | (bto >= OVERLAP_THRESH)
    posf = pos.astype(jnp.float32)

    g_cx = ((m_lo + m_hi) / 2.0 - a_cx) / (V0 * a_w)
    g_w = jnp.log(jnp.maximum((m_hi - m_lo) / a_w, 1e-10)) / V1
    d0 = lp_ref[0, 0] - g_cx
    d1 = lp_ref[0, 1] - g_w
    ad0 = jnp.abs(d0)
    ad1 = jnp.abs(d1)
    sl1 = (jnp.where(ad0 < 1.0, 0.5 * d0 * d0, ad0 - 0.5) +
           jnp.where(ad1 < 1.0, 0.5 * d1 * d1, ad1 - 0.5))
    loss_l = jnp.sum(sl1 * posf)

    x0 = cp_ref[0, 0]
    x1 = cp_ref[0, 1]
    m01 = jnp.maximum(x0, x1)
    lse = m01 + jnp.log(1.0 + jnp.exp(jnp.minimum(x0, x1) - m01))
    ce = lse - jnp.where(pos, x1, x0)
    ce_pos = jnp.sum(jnp.where(pos, ce, 0.0))

    num_pos = jnp.sum(pos.astype(jnp.int32))
    num_neg = jnp.minimum(NEG_POS_RATIO * num_pos, AR * AC - num_pos)

    proxy = jnp.where(pos, 0.0, ce)

    def topk_all():
        return jnp.sum(proxy)

    def topk_search():
        bits = jax.lax.bitcast_convert_type(proxy, jnp.int32)

        def bs_body(_, lo_hi):
            lo, hi = lo_hi
            mid = lo + (hi - lo) // 2
            cnt = jnp.sum((bits > mid).astype(jnp.int32))
            take = cnt >= num_neg
            return (jnp.where(take, mid, lo), jnp.where(take, hi, mid))

        _, kth = jax.lax.fori_loop(0, 31, bs_body,
                                   (jnp.int32(0), jnp.int32(2**31 - 1)))
        kth_val = jax.lax.bitcast_convert_type(kth, jnp.float32)
        gt = bits > kth
        sum_gt = jnp.sum(jnp.where(gt, proxy, 0.0))
        cnt_gt = jnp.sum(gt.astype(jnp.int32))
        return sum_gt + (num_neg - cnt_gt).astype(jnp.float32) * kth_val

    topk = jax.lax.cond(num_neg < AR * AC - num_pos, topk_search, topk_all)

    lane = jax.lax.broadcasted_iota(jnp.int32, (1, 128), 1)
    row = (jnp.where(lane == 0, loss_l, 0.0) +
           jnp.where(lane == 1, ce_pos, 0.0) +
           jnp.where(lane == 2, topk, 0.0) +
           jnp.where(lane == 3, num_pos.astype(jnp.float32), 0.0))
    out_ref[...] = row.reshape(1, 1, 128)


@jax.jit
def kernel(loc_pred, conf_pred, anchors, targets):
    B, A, _ = loc_pred.shape
    lp = loc_pred.transpose(0, 2, 1).reshape(B, 2, AR, AC)
    cp = conf_pred.transpose(0, 2, 1).reshape(B, 2, AR, AC)
    anch = anchors.T.reshape(2, AR, AC)

    parts = pl.pallas_call(
        _loss_kernel,
        grid=(B,),
        in_specs=[
            pl.BlockSpec((1, NOBJ, 3), lambda b: (b, 0, 0),
                         memory_space=pltpu.SMEM),
            pl.BlockSpec((2, AR, AC), lambda b: (0, 0, 0)),
            pl.BlockSpec((1, 2, AR, AC), lambda b: (b, 0, 0, 0)),
            pl.BlockSpec((1, 2, AR, AC), lambda b: (b, 0, 0, 0)),
        ],
        out_specs=pl.BlockSpec((1, 1, 128), lambda b: (b, 0, 0)),
        out_shape=jax.ShapeDtypeStruct((B, 1, 128), jnp.float32),
    )(targets, anch, lp, cp)

    loss_l = jnp.sum(parts[:, 0, 0])
    loss_c = jnp.sum(parts[:, 0, 1] + parts[:, 0, 2])
    total = jnp.sum(parts[:, 0, 3])
    return (loss_l / total, loss_c / total)
